# baseline (speedup 1.0000x reference)
import functools

import jax
import jax.numpy as jnp
from jax import lax
from jax.experimental import pallas as pl
from jax.experimental.pallas import tpu as pltpu
from jax.experimental.pallas import tpu_sc as plsc

NC = 2
NS = 16
NW = NC * NS


def _pick_chunk(ept: int) -> int:
    for c in range(10000, 7, -8):
        if ept % c == 0 and (ept // c) % 2 == 0:
            return c
    return 8

def _make_push_call(E: int, NPAD: int):
    assert E % NW == 0
    EPT = E // NW
    C = _pick_chunk(EPT)
    NCHUNK = EPT // C
    CHN = NPAD // NS
    assert NPAD % NS == 0 and CHN % 8 == 0

    mesh = plsc.VectorSubcoreMesh(core_axis_name="c", subcore_axis_name="s")

    @functools.partial(
        pl.kernel,
        out_type=jax.ShapeDtypeStruct((NC * NPAD,), jnp.float32),
        mesh=mesh,
        scratch_types=[
            pltpu.VMEM((C,), jnp.int32),
            pltpu.VMEM((C,), jnp.int32),
            pltpu.VMEM((C // 2,), jnp.int32),
            pltpu.VMEM((C // 2,), jnp.int32),
            pltpu.VMEM((C // 2,), jnp.int32),
            pltpu.VMEM((C // 2,), jnp.int32),
            pltpu.VMEM((C,), jnp.float32),
            pltpu.VMEM((C,), jnp.float32),
            pltpu.VMEM_SHARED((NPAD,), jnp.float32),
            pltpu.VMEM_SHARED((NPAD,), jnp.float32),
            pltpu.VMEM((CHN,), jnp.float32),
            pltpu.SemaphoreType.DMA,
            pltpu.SemaphoreType.DMA,
            pltpu.SemaphoreType.DMA,
            pltpu.SemaphoreType.DMA,
            pltpu.SemaphoreType.DMA,
            pltpu.SemaphoreType.DMA,
        ],
    )
    def push(src_hbm, tgt_hbm, val_hbm, zero_hbm, out_hbm,
             idx_s0, idx_s1, idx_t0a, idx_t0b, idx_t1a, idx_t1b,
             vals0, vals1, accum, vtab, stage,
             sem0, sem1, sem2, sem3, sem4, sem5):
        H = C // 2
        idx_s = (idx_s0, idx_s1)
        idx_t = ((idx_t0a, idx_t0b), (idx_t1a, idx_t1b))
        vals = (vals0, vals1)
        sem = (sem0, sem1)
        sem_s = ((sem2, sem3), (sem4, sem5))
        c = lax.axis_index("c")
        s = lax.axis_index("s")
        wid = c * NS + s

        pltpu.sync_copy(zero_hbm.at[pl.ds(s * CHN, CHN)], stage)
        pltpu.sync_copy(stage, accum.at[pl.ds(s * CHN, CHN)])
        pltpu.sync_copy(val_hbm.at[pl.ds(s * CHN, CHN)], stage)
        pltpu.sync_copy(stage, vtab.at[pl.ds(s * CHN, CHN)])
        plsc.subcore_barrier()

        ebase = wid * EPT

        pltpu.sync_copy(src_hbm.at[pl.ds(ebase, C)], idx_s[0])
        pltpu.sync_copy(tgt_hbm.at[pl.ds(ebase, H)], idx_t[0][0])
        pltpu.sync_copy(tgt_hbm.at[pl.ds(ebase + H, H)], idx_t[0][1])
        pltpu.async_copy(vtab.at[idx_s[0]], vals[0], sem[0])

        @pl.loop(0, NCHUNK, step=2)
        def _(g):
            for b in range(2):
                k = g + b
                nb = 1 - b

                @pl.when(k + 1 < NCHUNK)
                def _():
                    @pl.when(k >= 1)
                    def _():
                        for h in range(2):
                            pltpu.make_async_copy(
                                vals[nb].at[pl.ds(h * H, H)],
                                accum.at[idx_t[nb][h]],
                                sem_s[nb][h]).wait()
                    off2 = ebase + (k + 1) * C
                    pltpu.sync_copy(src_hbm.at[pl.ds(off2, C)], idx_s[nb])
                    pltpu.sync_copy(tgt_hbm.at[pl.ds(off2, H)], idx_t[nb][0])
                    pltpu.sync_copy(tgt_hbm.at[pl.ds(off2 + H, H)],
                                    idx_t[nb][1])
                    pltpu.async_copy(vtab.at[idx_s[nb]], vals[nb], sem[nb])

                pltpu.make_async_copy(vtab.at[idx_s[b]], vals[b],
                                      sem[b]).wait()
                for h in range(2):
                    pltpu.async_copy(vals[b].at[pl.ds(h * H, H)],
                                     accum.at[idx_t[b][h]], sem_s[b][h],
                                     add=True)

        for b in range(2):
            for h in range(2):
                pltpu.make_async_copy(vals[b].at[pl.ds(h * H, H)],
                                      accum.at[idx_t[b][h]],
                                      sem_s[b][h]).wait()
        plsc.subcore_barrier()
        pltpu.sync_copy(accum.at[pl.ds(s * CHN, CHN)], stage)
        pltpu.sync_copy(stage, out_hbm.at[pl.ds(c * NPAD + s * CHN, CHN)])

    return push


def kernel(source, target, init_vertex, iteration, vertex_num):
    N = init_vertex.shape[0]
    E = source.shape[0]
    NPAD = -(-N // (NS * 8)) * (NS * 8)

    src = source.astype(jnp.int32)
    tgt = target.astype(jnp.int32)

    push = _make_push_call(E, NPAD)

    zeros_pad = jnp.zeros((NPAD,), jnp.float32)
    ones_pad = zeros_pad.at[:N].set(1.0)

    deg_parts = push(src, src, ones_pad, zeros_pad)
    deg = deg_parts[:N] + deg_parts[NPAD:NPAD + N]
    mask = deg == 0.0
    degf = jnp.where(mask, 1.0, deg)

    V0 = init_vertex / jnp.sum(init_vertex)

    def cond_fun(carry):
        r, V_old, done = carry
        return jnp.logical_and(r < iteration, jnp.logical_not(done))

    def body_fun(carry):
        r, V_old, done = carry
        vtemp = jnp.where(mask, 0.0, V_old / degf)
        blind_sum = jnp.sum(jnp.where(mask, V_old, 0.0))
        vtemp_pad = jnp.concatenate([vtemp, jnp.zeros((NPAD - N,), jnp.float32)])
        parts = push(src, tgt, vtemp_pad, zeros_pad)
        V_new = parts[:N] + parts[NPAD:NPAD + N]
        V_new = V_new * 0.85 + (0.15 + blind_sum * 0.85) / vertex_num
        diff = jnp.sum(jnp.abs(V_new - V_old))
        return (r + 1, V_new, diff < 1e-07)

    carry = (jnp.int32(0), V0, jnp.bool_(False))
    _, V_out, _ = lax.while_loop(cond_fun, body_fun, carry)
    return V_out


# speedup vs baseline: 1.0676x; 1.0676x over previous
import functools

import jax
import jax.numpy as jnp
from jax import lax
from jax.experimental import pallas as pl
from jax.experimental.pallas import tpu as pltpu
from jax.experimental.pallas import tpu_sc as plsc

NC = 2
NS = 16
NW = NC * NS


def _pick_chunk(ept: int) -> int:
    for c in range(10000, 7, -8):
        if ept % c == 0 and (ept // c) % 2 == 0:
            return c
    return 8

def _make_push_call(E: int, NPAD: int):
    assert E % NW == 0
    EPT = E // NW
    C = _pick_chunk(EPT)
    NCHUNK = EPT // C
    CHN = NPAD // NS
    assert NPAD % NS == 0 and CHN % 8 == 0

    mesh = plsc.VectorSubcoreMesh(core_axis_name="c", subcore_axis_name="s")

    @functools.partial(
        pl.kernel,
        out_type=jax.ShapeDtypeStruct((NC * NPAD,), jnp.float32),
        mesh=mesh,
        scratch_types=[
            pltpu.VMEM((C,), jnp.int32),
            pltpu.VMEM((C,), jnp.int32),
            pltpu.VMEM((C,), jnp.int32),
            pltpu.VMEM((C,), jnp.int32),
            pltpu.VMEM((C,), jnp.float32),
            pltpu.VMEM((C,), jnp.float32),
            pltpu.VMEM_SHARED((NPAD,), jnp.float32),
            pltpu.VMEM_SHARED((NPAD,), jnp.float32),
            pltpu.VMEM((CHN,), jnp.float32),
            pltpu.SemaphoreType.DMA,
            pltpu.SemaphoreType.DMA,
            pltpu.SemaphoreType.DMA,
            pltpu.SemaphoreType.DMA,
        ],
    )
    def push(src_hbm, tgt_hbm, val_hbm, zero_hbm, out_hbm,
             idx_s0, idx_s1, idx_t0, idx_t1, vals0, vals1,
             accum, vtab, stage, sem0, sem1, sem2, sem3):
        idx_s = (idx_s0, idx_s1)
        idx_t = (idx_t0, idx_t1)
        vals = (vals0, vals1)
        sem = (sem0, sem1)
        sem_s = (sem2, sem3)
        c = lax.axis_index("c")
        s = lax.axis_index("s")
        wid = c * NS + s

        ebase = wid * EPT

        pltpu.sync_copy(val_hbm.at[pl.ds(s * CHN, CHN)], stage)
        pltpu.sync_copy(stage, vtab.at[pl.ds(s * CHN, CHN)])
        pltpu.sync_copy(src_hbm.at[pl.ds(ebase, C)], idx_s[0])
        pltpu.sync_copy(tgt_hbm.at[pl.ds(ebase, C)], idx_t[0])
        plsc.subcore_barrier()

        pltpu.async_copy(vtab.at[idx_s[0]], vals[0], sem[0])
        pltpu.sync_copy(zero_hbm.at[pl.ds(s * CHN, CHN)], stage)
        pltpu.sync_copy(stage, accum.at[pl.ds(s * CHN, CHN)])
        plsc.subcore_barrier()

        @pl.loop(0, NCHUNK, step=2)
        def _(g):
            for b in range(2):
                k = g + b
                nb = 1 - b

                @pl.when(k + 1 < NCHUNK)
                def _():
                    @pl.when(k >= 1)
                    def _():
                        pltpu.make_async_copy(vals[nb], accum.at[idx_t[nb]],
                                              sem_s[nb]).wait()
                    off2 = ebase + (k + 1) * C
                    pltpu.sync_copy(src_hbm.at[pl.ds(off2, C)], idx_s[nb])
                    pltpu.sync_copy(tgt_hbm.at[pl.ds(off2, C)], idx_t[nb])
                    pltpu.async_copy(vtab.at[idx_s[nb]], vals[nb], sem[nb])

                pltpu.make_async_copy(vtab.at[idx_s[b]], vals[b],
                                      sem[b]).wait()
                pltpu.async_copy(vals[b], accum.at[idx_t[b]], sem_s[b],
                                 add=True)

        for b in range(2):
            pltpu.make_async_copy(vals[b], accum.at[idx_t[b]],
                                  sem_s[b]).wait()
        plsc.subcore_barrier()
        pltpu.sync_copy(accum.at[pl.ds(s * CHN, CHN)], stage)
        pltpu.sync_copy(stage, out_hbm.at[pl.ds(c * NPAD + s * CHN, CHN)])

    return push


def kernel(source, target, init_vertex, iteration, vertex_num):
    N = init_vertex.shape[0]
    E = source.shape[0]
    NPAD = -(-N // (NS * 8)) * (NS * 8)

    src = source.astype(jnp.int32)
    tgt = target.astype(jnp.int32)

    push = _make_push_call(E, NPAD)

    zeros_pad = jnp.zeros((NPAD,), jnp.float32)
    ones_pad = zeros_pad.at[:N].set(1.0)

    deg_parts = push(src, src, ones_pad, zeros_pad)
    deg = deg_parts[:N] + deg_parts[NPAD:NPAD + N]
    mask = deg == 0.0
    degf = jnp.where(mask, 1.0, deg)

    V0 = init_vertex / jnp.sum(init_vertex)

    def cond_fun(carry):
        r, V_old, done = carry
        return jnp.logical_and(r < iteration, jnp.logical_not(done))

    def body_fun(carry):
        r, V_old, done = carry
        vtemp = jnp.where(mask, 0.0, V_old / degf)
        blind_sum = jnp.sum(jnp.where(mask, V_old, 0.0))
        vtemp_pad = jnp.concatenate([vtemp, jnp.zeros((NPAD - N,), jnp.float32)])
        parts = push(src, tgt, vtemp_pad, zeros_pad)
        V_new = parts[:N] + parts[NPAD:NPAD + N]
        V_new = V_new * 0.85 + (0.15 + blind_sum * 0.85) / vertex_num
        diff = jnp.sum(jnp.abs(V_new - V_old))
        return (r + 1, V_new, diff < 1e-07)

    carry = (jnp.int32(0), V0, jnp.bool_(False))
    _, V_out, _ = lax.while_loop(cond_fun, body_fun, carry)
    return V_out


# speedup vs baseline: 1.2483x; 1.1693x over previous
import functools

import jax
import jax.numpy as jnp
from jax import lax
from jax.experimental import pallas as pl
from jax.experimental.pallas import tpu as pltpu
from jax.experimental.pallas import tpu_sc as plsc

NC = 2
NS = 16
NW = NC * NS


def _pick_chunk(ept: int) -> int:
    for c in range(10000, 7, -8):
        if ept % c == 0 and (ept // c) % 2 == 0:
            return c
    return 8

def _make_push_call(E: int, NPAD: int):
    assert E % NW == 0
    EPT = E // NW
    C = _pick_chunk(EPT)
    NCHUNK = EPT // C
    CHN = NPAD // NS
    assert NPAD % NS == 0 and CHN % 8 == 0

    mesh = plsc.VectorSubcoreMesh(core_axis_name="c", subcore_axis_name="s")

    @functools.partial(
        pl.kernel,
        out_type=jax.ShapeDtypeStruct((NC * NPAD,), jnp.float32),
        mesh=mesh,
        scratch_types=[
            pltpu.VMEM((C,), jnp.int32),
            pltpu.VMEM((C,), jnp.int32),
            pltpu.VMEM((C,), jnp.int32),
            pltpu.VMEM((C,), jnp.int32),
            pltpu.VMEM((C,), jnp.float32),
            pltpu.VMEM((C,), jnp.float32),
            pltpu.VMEM_SHARED((NPAD,), jnp.float32),
            pltpu.VMEM_SHARED((NPAD,), jnp.float32),
            pltpu.VMEM((CHN,), jnp.float32),
            pltpu.SemaphoreType.DMA,
            pltpu.SemaphoreType.DMA,
            pltpu.SemaphoreType.DMA,
            pltpu.SemaphoreType.DMA,
            pltpu.SemaphoreType.DMA,
            pltpu.SemaphoreType.DMA,
        ],
    )
    def push(src_hbm, tgt_hbm, val_hbm, zero_hbm, out_hbm,
             idx_s0, idx_s1, idx_t0, idx_t1, vals0, vals1,
             accum, vtab, stage, sem0, sem1, sem2, sem3, sem4, sem5):
        idx_s = (idx_s0, idx_s1)
        idx_t = (idx_t0, idx_t1)
        vals = (vals0, vals1)
        sem = (sem0, sem1)
        sem_s = (sem2, sem3)
        sem_i = (sem4, sem5)
        c = lax.axis_index("c")
        s = lax.axis_index("s")
        wid = c * NS + s

        ebase = wid * EPT

        pltpu.sync_copy(val_hbm.at[pl.ds(s * CHN, CHN)], stage)
        pltpu.sync_copy(stage, vtab.at[pl.ds(s * CHN, CHN)])
        pltpu.async_copy(src_hbm.at[pl.ds(ebase, C)], idx_s[0], sem_i[0])
        pltpu.async_copy(tgt_hbm.at[pl.ds(ebase, C)], idx_t[0], sem_i[0])
        pltpu.sync_copy(zero_hbm.at[pl.ds(s * CHN, CHN)], stage)
        pltpu.sync_copy(stage, accum.at[pl.ds(s * CHN, CHN)])
        plsc.subcore_barrier()

        @pl.loop(0, NCHUNK, step=2)
        def _(g):
            for b in range(2):
                k = g + b
                nb = 1 - b

                pltpu.make_async_copy(src_hbm.at[pl.ds(ebase, C)],
                                      idx_s[b], sem_i[b]).wait()
                pltpu.make_async_copy(tgt_hbm.at[pl.ds(ebase, C)],
                                      idx_t[b], sem_i[b]).wait()

                @pl.when(k + 1 < NCHUNK)
                def _():
                    off2 = ebase + (k + 1) * C
                    pltpu.async_copy(src_hbm.at[pl.ds(off2, C)],
                                     idx_s[nb], sem_i[nb])
                    pltpu.async_copy(tgt_hbm.at[pl.ds(off2, C)],
                                     idx_t[nb], sem_i[nb])

                pltpu.async_copy(vtab.at[idx_s[b]], vals[b], sem[b])
                pltpu.make_async_copy(vtab.at[idx_s[b]], vals[b],
                                      sem[b]).wait()
                pltpu.async_copy(vals[b], accum.at[idx_t[b]], sem_s[b],
                                 add=True)
                pltpu.make_async_copy(vals[b], accum.at[idx_t[b]],
                                      sem_s[b]).wait()

        plsc.subcore_barrier()
        pltpu.sync_copy(accum.at[pl.ds(s * CHN, CHN)], stage)
        pltpu.sync_copy(stage, out_hbm.at[pl.ds(c * NPAD + s * CHN, CHN)])

    return push


def kernel(source, target, init_vertex, iteration, vertex_num):
    N = init_vertex.shape[0]
    E = source.shape[0]
    NPAD = -(-N // (NS * 8)) * (NS * 8)

    src = source.astype(jnp.int32)
    tgt = target.astype(jnp.int32)

    push = _make_push_call(E, NPAD)

    zeros_pad = jnp.zeros((NPAD,), jnp.float32)
    ones_pad = zeros_pad.at[:N].set(1.0)

    deg_parts = push(src, src, ones_pad, zeros_pad)
    deg = deg_parts[:N] + deg_parts[NPAD:NPAD + N]
    mask = deg == 0.0
    degf = jnp.where(mask, 1.0, deg)

    V0 = init_vertex / jnp.sum(init_vertex)

    def cond_fun(carry):
        r, V_old, done = carry
        return jnp.logical_and(r < iteration, jnp.logical_not(done))

    def body_fun(carry):
        r, V_old, done = carry
        vtemp = jnp.where(mask, 0.0, V_old / degf)
        blind_sum = jnp.sum(jnp.where(mask, V_old, 0.0))
        vtemp_pad = jnp.concatenate([vtemp, jnp.zeros((NPAD - N,), jnp.float32)])
        parts = push(src, tgt, vtemp_pad, zeros_pad)
        V_new = parts[:N] + parts[NPAD:NPAD + N]
        V_new = V_new * 0.85 + (0.15 + blind_sum * 0.85) / vertex_num
        diff = jnp.sum(jnp.abs(V_new - V_old))
        return (r + 1, V_new, diff < 1e-07)

    carry = (jnp.int32(0), V0, jnp.bool_(False))
    _, V_out, _ = lax.while_loop(cond_fun, body_fun, carry)
    return V_out


# speedup vs baseline: 1.3526x; 1.0835x over previous
import functools

import jax
import jax.numpy as jnp
from jax import lax
from jax.experimental import pallas as pl
from jax.experimental.pallas import tpu as pltpu
from jax.experimental.pallas import tpu_sc as plsc

NC = 2
NS = 16
NW = NC * NS


def _pick_chunk(ept: int) -> int:
    for c in range(10000, 7, -8):
        if ept % c == 0 and (ept // c) % 2 == 0:
            return c
    return 8

def _make_push_call(E: int, NPAD: int):
    assert E % NW == 0
    EPT = E // NW
    C = _pick_chunk(EPT)
    NCHUNK = EPT // C
    CHN = NPAD // NS
    assert NPAD % NS == 0 and CHN % 8 == 0

    mesh = plsc.VectorSubcoreMesh(core_axis_name="c", subcore_axis_name="s")

    @functools.partial(
        pl.kernel,
        out_type=jax.ShapeDtypeStruct((NC * NPAD,), jnp.float32),
        mesh=mesh,
        scratch_types=[
            pltpu.VMEM((C,), jnp.int32),
            pltpu.VMEM((C,), jnp.int32),
            pltpu.VMEM((C,), jnp.int32),
            pltpu.VMEM((C,), jnp.int32),
            pltpu.VMEM((C,), jnp.float32),
            pltpu.VMEM((C,), jnp.float32),
            pltpu.VMEM_SHARED((NPAD,), jnp.float32),
            pltpu.VMEM_SHARED((NPAD,), jnp.float32),
            pltpu.VMEM((CHN,), jnp.float32),
            pltpu.SemaphoreType.DMA,
            pltpu.SemaphoreType.DMA,
            pltpu.SemaphoreType.DMA,
            pltpu.SemaphoreType.DMA,
            pltpu.SemaphoreType.DMA,
            pltpu.SemaphoreType.DMA,
        ],
    )
    def push(src_hbm, tgt_hbm, val_hbm, zero_hbm, out_hbm,
             idx_s0, idx_s1, idx_t0, idx_t1, vals0, vals1,
             accum, vtab, stage, sem0, sem1, sem2, sem3, sem4, sem5):
        idx_s = (idx_s0, idx_s1)
        idx_t = (idx_t0, idx_t1)
        vals = (vals0, vals1)
        sem = (sem0, sem1)
        sem_s = (sem2, sem3)
        sem_i = (sem4, sem5)
        c = lax.axis_index("c")
        s = lax.axis_index("s")
        wid = c * NS + s

        ebase = wid * EPT

        pltpu.sync_copy(val_hbm.at[pl.ds(s * CHN, CHN)], stage)
        pltpu.sync_copy(stage, vtab.at[pl.ds(s * CHN, CHN)])
        pltpu.async_copy(src_hbm.at[pl.ds(ebase, C)], idx_s[0], sem_i[0])
        pltpu.async_copy(tgt_hbm.at[pl.ds(ebase, C)], idx_t[0], sem_i[0])
        pltpu.sync_copy(zero_hbm.at[pl.ds(s * CHN, CHN)], stage)
        pltpu.sync_copy(stage, accum.at[pl.ds(s * CHN, CHN)])
        plsc.subcore_barrier()

        @pl.loop(0, NCHUNK, step=2)
        def _(g):
            for b in range(2):
                k = g + b
                nb = 1 - b

                pltpu.make_async_copy(src_hbm.at[pl.ds(ebase, C)],
                                      idx_s[b], sem_i[b]).wait()
                pltpu.make_async_copy(tgt_hbm.at[pl.ds(ebase, C)],
                                      idx_t[b], sem_i[b]).wait()

                @pl.when(k + 1 < NCHUNK)
                def _():
                    off2 = ebase + (k + 1) * C
                    pltpu.async_copy(src_hbm.at[pl.ds(off2, C)],
                                     idx_s[nb], sem_i[nb])
                    pltpu.async_copy(tgt_hbm.at[pl.ds(off2, C)],
                                     idx_t[nb], sem_i[nb])

                pltpu.async_copy(vtab.at[idx_s[b]], vals[b], sem[b])
                pltpu.make_async_copy(vtab.at[idx_s[b]], vals[b],
                                      sem[b]).wait()
                plsc.subcore_barrier()
                pltpu.async_copy(vals[b], accum.at[idx_t[b]], sem_s[b],
                                 add=True)
                pltpu.make_async_copy(vals[b], accum.at[idx_t[b]],
                                      sem_s[b]).wait()
                plsc.subcore_barrier()

        plsc.subcore_barrier()
        pltpu.sync_copy(accum.at[pl.ds(s * CHN, CHN)], stage)
        pltpu.sync_copy(stage, out_hbm.at[pl.ds(c * NPAD + s * CHN, CHN)])

    return push


def kernel(source, target, init_vertex, iteration, vertex_num):
    N = init_vertex.shape[0]
    E = source.shape[0]
    NPAD = -(-N // (NS * 8)) * (NS * 8)

    src = source.astype(jnp.int32)
    tgt = target.astype(jnp.int32)

    push = _make_push_call(E, NPAD)

    zeros_pad = jnp.zeros((NPAD,), jnp.float32)
    ones_pad = zeros_pad.at[:N].set(1.0)

    deg_parts = push(src, src, ones_pad, zeros_pad)
    deg = deg_parts[:N] + deg_parts[NPAD:NPAD + N]
    mask = deg == 0.0
    degf = jnp.where(mask, 1.0, deg)

    V0 = init_vertex / jnp.sum(init_vertex)

    def cond_fun(carry):
        r, V_old, done = carry
        return jnp.logical_and(r < iteration, jnp.logical_not(done))

    def body_fun(carry):
        r, V_old, done = carry
        vtemp = jnp.where(mask, 0.0, V_old / degf)
        blind_sum = jnp.sum(jnp.where(mask, V_old, 0.0))
        vtemp_pad = jnp.concatenate([vtemp, jnp.zeros((NPAD - N,), jnp.float32)])
        parts = push(src, tgt, vtemp_pad, zeros_pad)
        V_new = parts[:N] + parts[NPAD:NPAD + N]
        V_new = V_new * 0.85 + (0.15 + blind_sum * 0.85) / vertex_num
        diff = jnp.sum(jnp.abs(V_new - V_old))
        return (r + 1, V_new, diff < 1e-07)

    carry = (jnp.int32(0), V0, jnp.bool_(False))
    _, V_out, _ = lax.while_loop(cond_fun, body_fun, carry)
    return V_out
